# baseline (speedup 1.0000x reference)
import jax
import jax.numpy as jnp
from jax.experimental import pallas as pl

B, V, D_IN, D_H, D_OUT = 1024, 128, 256, 512, 256


def _mlp_kernel(x_ref, w0_ref, b0_ref, w1_ref, b1_ref, w2_ref, b2_ref, out_ref):
    bf = jnp.bfloat16
    dn = (((1,), (1,)), ((), ()))
    xv = x_ref[:, 0, 0, :].astype(bf)
    h = jax.lax.dot_general(xv, w0_ref[0].astype(bf), dn,
                            preferred_element_type=jnp.float32)
    h = jnp.maximum(h + b0_ref[0], 0.0).astype(bf)
    h = jax.lax.dot_general(h, w1_ref[0].astype(bf), dn,
                            preferred_element_type=jnp.float32)
    h = jnp.maximum(h + b1_ref[0], 0.0).astype(bf)
    o = jax.lax.dot_general(h, w2_ref[0].astype(bf), dn,
                            preferred_element_type=jnp.float32)
    out_ref[:, 0, 0, :] = o + b2_ref[0]


def kernel(x, W0, b0, W1, b1, W2, b2):
    out = pl.pallas_call(
        _mlp_kernel,
        grid=(V,),
        in_specs=[
            pl.BlockSpec((B, 1, 1, D_IN), lambda v: (0, v, 0, 0)),
            pl.BlockSpec((1, D_H, D_IN), lambda v: (v, 0, 0)),
            pl.BlockSpec((1, 1, D_H), lambda v: (v, 0, 0)),
            pl.BlockSpec((1, D_H, D_H), lambda v: (v, 0, 0)),
            pl.BlockSpec((1, 1, D_H), lambda v: (v, 0, 0)),
            pl.BlockSpec((1, D_OUT, D_H), lambda v: (v, 0, 0)),
            pl.BlockSpec((1, 1, D_OUT), lambda v: (v, 0, 0)),
        ],
        out_specs=pl.BlockSpec((B, 1, 1, D_OUT), lambda v: (0, v, 0, 0)),
        out_shape=jax.ShapeDtypeStruct((B, V, 1, D_OUT), jnp.float32),
    )(
        x.reshape(B, V, 1, D_IN),
        W0,
        b0.reshape(V, 1, D_H),
        W1,
        b1.reshape(V, 1, D_H),
        W2,
        b2.reshape(V, 1, D_OUT),
    )
    return out.reshape(B, V, D_OUT)


# speedup vs baseline: 1.2308x; 1.2308x over previous
import jax
import jax.numpy as jnp
from jax.experimental import pallas as pl

B, V, D_IN, D_H, D_OUT = 1024, 128, 256, 512, 256
VT = 2


def _mlp_kernel(x_ref, w0_ref, b0_ref, w1_ref, b1_ref, w2_ref, b2_ref, out_ref):
    dn = (((1,), (1,)), ((), ()))
    for i in range(VT):
        xv = x_ref[:, i, 0, :]
        h = jax.lax.dot_general(xv, w0_ref[i], dn, preferred_element_type=jnp.float32)
        h = jnp.maximum(h + b0_ref[i], 0.0)
        h = jax.lax.dot_general(h, w1_ref[i], dn, preferred_element_type=jnp.float32)
        h = jnp.maximum(h + b1_ref[i], 0.0)
        o = jax.lax.dot_general(h, w2_ref[i], dn, preferred_element_type=jnp.float32)
        out_ref[:, i, 0, :] = o + b2_ref[i]


def kernel(x, W0, b0, W1, b1, W2, b2):
    out = pl.pallas_call(
        _mlp_kernel,
        grid=(V // VT,),
        in_specs=[
            pl.BlockSpec((B, VT, 1, D_IN), lambda v: (0, v, 0, 0)),
            pl.BlockSpec((VT, D_H, D_IN), lambda v: (v, 0, 0)),
            pl.BlockSpec((VT, 1, D_H), lambda v: (v, 0, 0)),
            pl.BlockSpec((VT, D_H, D_H), lambda v: (v, 0, 0)),
            pl.BlockSpec((VT, 1, D_H), lambda v: (v, 0, 0)),
            pl.BlockSpec((VT, D_OUT, D_H), lambda v: (v, 0, 0)),
            pl.BlockSpec((VT, 1, D_OUT), lambda v: (v, 0, 0)),
        ],
        out_specs=pl.BlockSpec((B, VT, 1, D_OUT), lambda v: (0, v, 0, 0)),
        out_shape=jax.ShapeDtypeStruct((B, V, 1, D_OUT), jnp.float32),
    )(
        x.reshape(B, V, 1, D_IN),
        W0,
        b0.reshape(V, 1, D_H),
        W1,
        b1.reshape(V, 1, D_H),
        W2,
        b2.reshape(V, 1, D_OUT),
    )
    return out.reshape(B, V, D_OUT)


# speedup vs baseline: 1.2669x; 1.0293x over previous
import jax
import jax.numpy as jnp
from jax.experimental import pallas as pl

B, V, D_IN, D_H, D_OUT = 1024, 128, 256, 512, 256
VT = 4


def _mlp_kernel(x_ref, w0_ref, b0_ref, w1_ref, b1_ref, w2_ref, b2_ref, out_ref):
    dn = (((1,), (1,)), ((), ()))
    for i in range(VT):
        xv = x_ref[:, i, 0, :]
        h = jax.lax.dot_general(xv, w0_ref[i], dn, preferred_element_type=jnp.float32)
        h = jnp.maximum(h + b0_ref[i], 0.0)
        h = jax.lax.dot_general(h, w1_ref[i], dn, preferred_element_type=jnp.float32)
        h = jnp.maximum(h + b1_ref[i], 0.0)
        o = jax.lax.dot_general(h, w2_ref[i], dn, preferred_element_type=jnp.float32)
        out_ref[:, i, 0, :] = o + b2_ref[i]


def kernel(x, W0, b0, W1, b1, W2, b2):
    out = pl.pallas_call(
        _mlp_kernel,
        grid=(V // VT,),
        in_specs=[
            pl.BlockSpec((B, VT, 1, D_IN), lambda v: (0, v, 0, 0)),
            pl.BlockSpec((VT, D_H, D_IN), lambda v: (v, 0, 0)),
            pl.BlockSpec((VT, 1, D_H), lambda v: (v, 0, 0)),
            pl.BlockSpec((VT, D_H, D_H), lambda v: (v, 0, 0)),
            pl.BlockSpec((VT, 1, D_H), lambda v: (v, 0, 0)),
            pl.BlockSpec((VT, D_OUT, D_H), lambda v: (v, 0, 0)),
            pl.BlockSpec((VT, 1, D_OUT), lambda v: (v, 0, 0)),
        ],
        out_specs=pl.BlockSpec((B, VT, 1, D_OUT), lambda v: (0, v, 0, 0)),
        out_shape=jax.ShapeDtypeStruct((B, V, 1, D_OUT), jnp.float32),
    )(
        x.reshape(B, V, 1, D_IN),
        W0,
        b0.reshape(V, 1, D_H),
        W1,
        b1.reshape(V, 1, D_H),
        W2,
        b2.reshape(V, 1, D_OUT),
    )
    return out.reshape(B, V, D_OUT)
